# baseline (speedup 1.0000x reference)
import functools

import jax
import jax.numpy as jnp
from jax import lax
from jax.experimental import pallas as pl
from jax.experimental.pallas import tpu as pltpu
from jax.experimental.pallas import tpu_sc as plsc

_NUM_CORES = 2
_NUM_SUBCORES = 16
_NUM_WORKERS = _NUM_CORES * _NUM_SUBCORES
_CHUNK = 128
_GRAN = 8
_D = 300
_KPG = 38
_PITCH = _KPG * _GRAN


@functools.partial(jax.jit, static_argnames=("n_chunks_per_worker",))
def _sc_gather(idxe, idxw, table8, *, n_chunks_per_worker):
    n_rows = _NUM_WORKERS * n_chunks_per_worker * _CHUNK
    mesh = plsc.VectorSubcoreMesh(
        core_axis_name="c",
        subcore_axis_name="s",
        num_cores=_NUM_CORES,
        num_subcores=_NUM_SUBCORES,
    )

    @functools.partial(
        pl.kernel,
        out_type=jax.ShapeDtypeStruct((n_rows * _D,), jnp.float32),
        mesh=mesh,
        scratch_types=[
            pltpu.VMEM((_KPG * _CHUNK,), jnp.int32),
            pltpu.VMEM((_KPG * _CHUNK + _GRAN, _GRAN), jnp.float32),
            pltpu.VMEM((_CHUNK * _D + 16,), jnp.float32),
            pltpu.VMEM((_CHUNK,), jnp.int32),
            pltpu.SemaphoreType.DMA,
        ],
        compiler_params=pltpu.CompilerParams(
            use_tc_tiling_on_sc=False, needs_layout_passes=False
        ),
    )
    def body(
        table8_hbm, idxe_hbm, idxw_hbm, out_hbm, ie_v, buf_v, stg_v, idx_v, gsem
    ):
        wid = lax.axis_index("s") * _NUM_CORES + lax.axis_index("c")
        lane = lax.iota(jnp.int32, 16)
        r0, c0 = lane >> 3, lane & 7
        r4, c4 = (lane + 4) >> 3, (lane + 4) & 7

        @pl.loop(0, n_chunks_per_worker)
        def _chunk(c):
            cc = wid * n_chunks_per_worker + c
            pltpu.sync_copy(idxe_hbm.at[cc], ie_v)
            pltpu.sync_copy(idxw_hbm.at[cc], idx_v)
            pltpu.async_copy(
                table8_hbm.at[ie_v], buf_v.at[pl.ds(0, _KPG * _CHUNK)], gsem
            ).wait()

            @pl.loop(0, _CHUNK)
            def _row(p):
                rv = plsc.load_gather(idx_v, [jnp.full((16,), p, jnp.int32)])
                odd = (rv & 1) == 1
                rsel = jnp.where(odd, r4, r0) + p * _KPG
                csel = jnp.where(odd, c4, c0)
                dst0 = p * _D
                for g in range(19):
                    v = plsc.load_gather(buf_v, [rsel + 2 * g, csel])
                    stg_v[pl.ds(dst0 + g * 16, 16)] = v

            pltpu.sync_copy(
                stg_v.at[pl.ds(0, _CHUNK * _D)],
                out_hbm.at[pl.ds(cc * _CHUNK * _D, _CHUNK * _D)],
            )

    return body(table8, idxe, idxw)


def kernel(batch, table):
    b, s = batch.shape
    v, d = table.shape
    assert d == _D
    n = b * s
    assert n % (_NUM_WORKERS * _CHUNK) == 0
    n_chunks_per_worker = n // (_NUM_WORKERS * _CHUNK)
    n_chunks = n // _CHUNK

    flat_idx = batch.astype(jnp.int32).reshape(n)
    base = (flat_idx * _D) >> 3
    idxe = (
        base[:, None] + jnp.arange(_KPG, dtype=jnp.int32)[None, :]
    ).reshape(n_chunks, _KPG * _CHUNK)
    idxw = flat_idx.reshape(n_chunks, _CHUNK)
    table8 = table.reshape(v * d // _GRAN, _GRAN)

    out = _sc_gather(idxe, idxw, table8, n_chunks_per_worker=n_chunks_per_worker)
    return out.reshape(b, s, d)


# speedup vs baseline: 2.2103x; 2.2103x over previous
import functools

import jax
import jax.numpy as jnp
from jax import lax
from jax.experimental import pallas as pl
from jax.experimental.pallas import tpu as pltpu
from jax.experimental.pallas import tpu_sc as plsc

_NUM_CORES = 2
_NUM_SUBCORES = 16
_NUM_WORKERS = _NUM_CORES * _NUM_SUBCORES
_CHUNK = 128
_D = 300
_DPAD = 384
_BR = 2048


def _tc_retile_kernel(t_ref, o_ref):
    x = t_ref[...]
    xp = jnp.concatenate(
        [x, jnp.zeros((_DPAD - _D, _BR), jnp.float32)], axis=0
    )
    o_ref[...] = jnp.transpose(xp, (1, 0))


@jax.jit
def _tc_retile(table_t):
    v = table_t.shape[1]
    grid = (v + _BR - 1) // _BR
    return pl.pallas_call(
        _tc_retile_kernel,
        out_shape=jax.ShapeDtypeStruct((v, _DPAD), jnp.float32),
        grid=(grid,),
        in_specs=[pl.BlockSpec((_D, _BR), lambda b: (0, b))],
        out_specs=pl.BlockSpec((_BR, _DPAD), lambda b: (b, 0)),
    )(table_t)


@functools.partial(jax.jit, static_argnames=("n_chunks_per_worker",))
def _sc_gather(idx, padded, *, n_chunks_per_worker):
    n_rows = _NUM_WORKERS * n_chunks_per_worker * _CHUNK
    mesh = plsc.VectorSubcoreMesh(
        core_axis_name="c",
        subcore_axis_name="s",
        num_cores=_NUM_CORES,
        num_subcores=_NUM_SUBCORES,
    )

    @functools.partial(
        pl.kernel,
        out_type=jax.ShapeDtypeStruct((n_rows * _D,), jnp.float32),
        mesh=mesh,
        scratch_types=[
            pltpu.VMEM((_CHUNK,), jnp.int32),
            pltpu.VMEM((_CHUNK, _DPAD), jnp.float32),
            pltpu.VMEM((_CHUNK * _D + 16,), jnp.float32),
            pltpu.SemaphoreType.DMA,
        ],
        compiler_params=pltpu.CompilerParams(
            use_tc_tiling_on_sc=False, needs_layout_passes=False
        ),
    )
    def body(padded_hbm, idx_hbm, out_hbm, idx_v, buf_v, stg_v, gsem):
        wid = lax.axis_index("s") * _NUM_CORES + lax.axis_index("c")

        @pl.loop(0, n_chunks_per_worker)
        def _chunk(c):
            cc = wid * n_chunks_per_worker + c
            pltpu.sync_copy(idx_hbm.at[cc], idx_v)
            pltpu.async_copy(padded_hbm.at[idx_v], buf_v, gsem).wait()

            @pl.loop(0, _CHUNK)
            def _row(p):
                dst0 = p * _D
                for g in range(19):
                    stg_v[pl.ds(dst0 + g * 16, 16)] = buf_v[p, pl.ds(g * 16, 16)]

            pltpu.sync_copy(
                stg_v.at[pl.ds(0, _CHUNK * _D)],
                out_hbm.at[pl.ds(cc * _CHUNK * _D, _CHUNK * _D)],
            )

    return body(padded, idx)


def kernel(batch, table):
    b, s = batch.shape
    v, d = table.shape
    assert d == _D
    n = b * s
    assert n % (_NUM_WORKERS * _CHUNK) == 0
    n_chunks_per_worker = n // (_NUM_WORKERS * _CHUNK)
    n_chunks = n // _CHUNK

    padded = _tc_retile(table.T)
    idx = batch.astype(jnp.int32).reshape(n_chunks, _CHUNK)
    out = _sc_gather(idx, padded, n_chunks_per_worker=n_chunks_per_worker)
    return out.reshape(b, s, d)


# speedup vs baseline: 2.2553x; 1.0204x over previous
import functools

import jax
import jax.numpy as jnp
from jax import lax
from jax.experimental import pallas as pl
from jax.experimental.pallas import tpu as pltpu
from jax.experimental.pallas import tpu_sc as plsc

_NUM_CORES = 2
_NUM_SUBCORES = 16
_NUM_WORKERS = _NUM_CORES * _NUM_SUBCORES
_CHUNK = 128
_D = 300
_DPAD = 384
_BR = 4096


def _tc_retile_kernel(t_ref, o_ref):
    x = t_ref[...]
    o_ref[:, : _D] = jnp.transpose(x, (1, 0))


@jax.jit
def _tc_retile(table_t):
    v = table_t.shape[1]
    grid = (v + _BR - 1) // _BR
    return pl.pallas_call(
        _tc_retile_kernel,
        out_shape=jax.ShapeDtypeStruct((v, _DPAD), jnp.float32),
        grid=(grid,),
        in_specs=[pl.BlockSpec((_D, _BR), lambda b: (0, b))],
        out_specs=pl.BlockSpec((_BR, _DPAD), lambda b: (b, 0)),
    )(table_t)


@functools.partial(jax.jit, static_argnames=("n_chunks_per_worker",))
def _sc_gather(idx, padded, *, n_chunks_per_worker):
    n_rows = _NUM_WORKERS * n_chunks_per_worker * _CHUNK
    mesh = plsc.VectorSubcoreMesh(
        core_axis_name="c",
        subcore_axis_name="s",
        num_cores=_NUM_CORES,
        num_subcores=_NUM_SUBCORES,
    )

    @functools.partial(
        pl.kernel,
        out_type=jax.ShapeDtypeStruct((n_rows * _D,), jnp.float32),
        mesh=mesh,
        scratch_types=[
            pltpu.VMEM((_CHUNK,), jnp.int32),
            pltpu.VMEM((_CHUNK, _DPAD), jnp.float32),
            pltpu.VMEM((_CHUNK * _D + 16,), jnp.float32),
            pltpu.SemaphoreType.DMA,
        ],
        compiler_params=pltpu.CompilerParams(
            use_tc_tiling_on_sc=False, needs_layout_passes=False
        ),
    )
    def body(padded_hbm, idx_hbm, out_hbm, idx_v, buf_v, stg_v, gsem):
        wid = lax.axis_index("s") * _NUM_CORES + lax.axis_index("c")

        @pl.loop(0, n_chunks_per_worker)
        def _chunk(c):
            cc = wid * n_chunks_per_worker + c
            pltpu.sync_copy(idx_hbm.at[cc], idx_v)
            pltpu.async_copy(padded_hbm.at[idx_v], buf_v, gsem).wait()

            @pl.loop(0, _CHUNK)
            def _row(p):
                dst0 = p * _D
                for g in range(19):
                    stg_v[pl.ds(dst0 + g * 16, 16)] = buf_v[p, pl.ds(g * 16, 16)]

            pltpu.sync_copy(
                stg_v.at[pl.ds(0, _CHUNK * _D)],
                out_hbm.at[pl.ds(cc * _CHUNK * _D, _CHUNK * _D)],
            )

    return body(padded, idx)


def kernel(batch, table):
    b, s = batch.shape
    v, d = table.shape
    assert d == _D
    n = b * s
    assert n % (_NUM_WORKERS * _CHUNK) == 0
    n_chunks_per_worker = n // (_NUM_WORKERS * _CHUNK)
    n_chunks = n // _CHUNK

    padded = _tc_retile(table.T)
    idx = batch.astype(jnp.int32).reshape(n_chunks, _CHUNK)
    out = _sc_gather(idx, padded, n_chunks_per_worker=n_chunks_per_worker)
    return out.reshape(b, s, d)


# speedup vs baseline: 2.3440x; 1.0393x over previous
import functools

import jax
import jax.numpy as jnp
from jax import lax
from jax.experimental import pallas as pl
from jax.experimental.pallas import tpu as pltpu
from jax.experimental.pallas import tpu_sc as plsc

_NUM_CORES = 2
_NUM_SUBCORES = 16
_NUM_WORKERS = _NUM_CORES * _NUM_SUBCORES
_CHUNK = 100
_D = 300
_DPAD = 384
_BR = 8192


def _tc_retile_kernel(t_ref, o_ref):
    x = t_ref[...]
    o_ref[:, : _D] = jnp.transpose(x, (1, 0))


@jax.jit
def _tc_retile(table_t):
    v = table_t.shape[1]
    grid = (v + _BR - 1) // _BR
    return pl.pallas_call(
        _tc_retile_kernel,
        out_shape=jax.ShapeDtypeStruct((v, _DPAD), jnp.float32),
        grid=(grid,),
        in_specs=[pl.BlockSpec((_D, _BR), lambda b: (0, b))],
        out_specs=pl.BlockSpec((_BR, _DPAD), lambda b: (b, 0)),
    )(table_t)


@functools.partial(jax.jit, static_argnames=("n_chunks_per_worker",))
def _sc_gather(idx, padded, *, n_chunks_per_worker):
    n_rows = _NUM_WORKERS * n_chunks_per_worker * _CHUNK
    mesh = plsc.VectorSubcoreMesh(
        core_axis_name="c",
        subcore_axis_name="s",
        num_cores=_NUM_CORES,
        num_subcores=_NUM_SUBCORES,
    )

    @functools.partial(
        pl.kernel,
        out_type=jax.ShapeDtypeStruct((n_rows * _D,), jnp.float32),
        mesh=mesh,
        scratch_types=[
            pltpu.VMEM((2, _CHUNK), jnp.int32),
            pltpu.VMEM((_CHUNK, _DPAD), jnp.float32),
            pltpu.VMEM((_CHUNK, _DPAD), jnp.float32),
            pltpu.VMEM((_CHUNK * _D + 16,), jnp.float32),
            pltpu.SemaphoreType.DMA,
            pltpu.SemaphoreType.DMA,
        ],
        compiler_params=pltpu.CompilerParams(
            use_tc_tiling_on_sc=False, needs_layout_passes=False
        ),
    )
    def body(padded_hbm, idx_hbm, out_hbm, idx_v, buf0, buf1, stg_v, sem0, sem1):
        wid = lax.axis_index("s") * _NUM_CORES + lax.axis_index("c")
        cc0 = wid * n_chunks_per_worker
        bufs = (buf0, buf1)
        sems = (sem0, sem1)

        def compact_and_store(sl, cc):
            @pl.loop(0, _CHUNK)
            def _row(p):
                dst0 = p * _D
                for g in range(19):
                    stg_v[pl.ds(dst0 + g * 16, 16)] = bufs[sl][p, pl.ds(g * 16, 16)]

            pltpu.sync_copy(
                stg_v.at[pl.ds(0, _CHUNK * _D)],
                out_hbm.at[pl.ds(cc * _CHUNK * _D, _CHUNK * _D)],
            )

        pltpu.sync_copy(idx_hbm.at[cc0], idx_v.at[0])
        pltpu.async_copy(padded_hbm.at[idx_v.at[0]], buf0, sem0)

        @pl.loop(0, n_chunks_per_worker, step=2)
        def _chunk(c):
            for sl in (0, 1):
                cc = c + sl

                @pl.when(cc + 1 < n_chunks_per_worker)
                def _prefetch():
                    pltpu.sync_copy(idx_hbm.at[cc0 + cc + 1], idx_v.at[1 - sl])
                    pltpu.async_copy(
                        padded_hbm.at[idx_v.at[1 - sl]], bufs[1 - sl], sems[1 - sl]
                    )

                pltpu.make_async_copy(padded_hbm.at[idx_v.at[sl]], bufs[sl], sems[sl]).wait()
                compact_and_store(sl, cc0 + cc)

    return body(padded, idx)


def kernel(batch, table):
    b, s = batch.shape
    v, d = table.shape
    assert d == _D
    n = b * s
    assert n % (_NUM_WORKERS * _CHUNK) == 0
    n_chunks_per_worker = n // (_NUM_WORKERS * _CHUNK)
    n_chunks = n // _CHUNK

    padded = _tc_retile(table.T)
    idx = batch.astype(jnp.int32).reshape(n_chunks, _CHUNK)
    out = _sc_gather(idx, padded, n_chunks_per_worker=n_chunks_per_worker)
    return out.reshape(b, s, d)
